# baseline (speedup 1.0000x reference)
import jax
import jax.numpy as jnp
from jax.experimental import pallas as pl
from jax.experimental.pallas import tpu as pltpu

N = 16384
IN_DIM = 512
H1 = 256
H2 = 128
OUT_DIM = 32

BLOCK = 2048
G = N // (2 * BLOCK)


def _mlp(x_ref, w1, b1, w2, b2, w3, b3):
    h = jnp.dot(x_ref[...].astype(jnp.bfloat16), w1,
                preferred_element_type=jnp.float32)
    h = jnp.maximum(h + b1, 0.0)
    h = jnp.dot(h.astype(jnp.bfloat16), w2, preferred_element_type=jnp.float32)
    h = jnp.maximum(h + b2, 0.0)
    h = jnp.dot(h.astype(jnp.bfloat16), w3, preferred_element_type=jnp.float32)
    return h + b3


def _mlp_block(xa_ref, xb_ref, w1_ref, b1_ref, w2_ref, b2_ref, w3_ref, b3_ref,
               out_ref):
    w1 = w1_ref[...].astype(jnp.bfloat16)
    w2 = w2_ref[...].astype(jnp.bfloat16)
    w3 = w3_ref[...].astype(jnp.bfloat16)
    b1 = b1_ref[...]
    b2 = b2_ref[...]
    b3 = b3_ref[...]
    out_ref[:BLOCK] = _mlp(xa_ref, w1, b1, w2, b2, w3, b3)
    out_ref[BLOCK:] = _mlp(xb_ref, w1, b1, w2, b2, w3, b3)


def kernel(x, W1, b1, W2, b2, W3, b3):
    b1r = b1.reshape(1, H1)
    b2r = b2.reshape(1, H2)
    b3r = b3.reshape(1, OUT_DIM)
    return pl.pallas_call(
        _mlp_block,
        grid=(G,),
        in_specs=[
            pl.BlockSpec((BLOCK, IN_DIM), lambda i: (2 * i, 0)),
            pl.BlockSpec((BLOCK, IN_DIM), lambda i: (2 * i + 1, 0)),
            pl.BlockSpec((IN_DIM, H1), lambda i: (0, 0)),
            pl.BlockSpec((1, H1), lambda i: (0, 0)),
            pl.BlockSpec((H1, H2), lambda i: (0, 0)),
            pl.BlockSpec((1, H2), lambda i: (0, 0)),
            pl.BlockSpec((H2, OUT_DIM), lambda i: (0, 0)),
            pl.BlockSpec((1, OUT_DIM), lambda i: (0, 0)),
        ],
        out_specs=pl.BlockSpec((2 * BLOCK, OUT_DIM), lambda i: (i, 0)),
        out_shape=jax.ShapeDtypeStruct((N, OUT_DIM), jnp.float32),
        compiler_params=pltpu.CompilerParams(
            dimension_semantics=("arbitrary",),
        ),
    )(x, x, W1, b1r, W2, b2r, W3, b3r)
